# baseline (speedup 1.0000x reference)
import functools

import jax
import jax.numpy as jnp
from jax import lax
from jax.experimental import pallas as pl
from jax.experimental.pallas import tpu as pltpu
from jax.experimental.pallas import tpu_sc as plsc

_B, _D, _Q, _L = 32, 8, 15, 500
_V, _E, _NB = 100000, 300, 5
_EP = 304
_NW = 32
_DPW = (_B * _D * _L) // _NW
_CH = 80
_NCH = _DPW // _CH
_QPW = 16


@functools.cache
def _sc_gather_build():
    mesh = plsc.VectorSubcoreMesh(
        core_axis_name="c", subcore_axis_name="s", num_cores=2)

    @functools.partial(
        pl.kernel,
        mesh=mesh,
        out_type=(
            jax.ShapeDtypeStruct((_B * _D * _L, _EP), jnp.float32),
            jax.ShapeDtypeStruct((_B * _QPW, _EP), jnp.float32),
        ),
        scratch_types=[
            pltpu.VMEM((_DPW,), jnp.int32),
            pltpu.VMEM((_QPW,), jnp.int32),
            pltpu.VMEM((_CH, _EP), jnp.float32),
            pltpu.SemaphoreType.DMA,
        ],
        compiler_params=pltpu.CompilerParams(use_tc_tiling_on_sc=False),
    )
    def sc_gather(emb_hbm, didx_hbm, qidx_hbm, dout_hbm, qout_hbm,
                  didx_v, qidx_v, buf, sem):
        wid = lax.axis_index("s") * 2 + lax.axis_index("c")
        dbase = wid * _DPW
        qbase = wid * _QPW
        pltpu.sync_copy(didx_hbm.at[pl.ds(dbase, _DPW)], didx_v)
        pltpu.sync_copy(qidx_hbm.at[pl.ds(qbase, _QPW)], qidx_v)
        pltpu.async_copy(emb_hbm.at[qidx_v], buf.at[pl.ds(0, _QPW)], sem).wait()
        pltpu.sync_copy(buf.at[pl.ds(0, _QPW)], qout_hbm.at[pl.ds(qbase, _QPW)])

        def chunk(k, carry):
            off = k * _CH
            pltpu.async_copy(
                emb_hbm.at[didx_v.at[pl.ds(off, _CH)]], buf, sem).wait()
            pltpu.sync_copy(buf, dout_hbm.at[pl.ds(dbase + off, _CH)])
            return carry

        lax.fori_loop(0, _NCH, chunk, 0)

    return sc_gather


def _tc_body(d_ref, q_ref, gw_ref, pp_ref, out_ref):
    d = d_ref[0]
    q = q_ref[0, 0:_Q, :]
    dots = lax.dot_general(
        d, q, (((1,), (1,)), ((), ())),
        preferred_element_type=jnp.float32,
        precision=lax.Precision.DEFAULT)
    dn = jnp.sqrt(jnp.sum(d * d, axis=1, keepdims=True))
    qn = jnp.sqrt(jnp.sum(q * q, axis=1))[None, :]
    denom = jnp.maximum(dn * qn, 1e-8)
    cos = jnp.clip(dots / denom, -1.0, 1.0)
    cnt = [jnp.sum((cos >= t).astype(jnp.float32), axis=0)
           for t in (-0.5, 0.0, 0.5, 1.0)]
    def _r(x):
        return x.astype(jnp.bfloat16).astype(jnp.float32)

    h = [jnp.float32(_L) - cnt[0], cnt[0] - cnt[1], cnt[1] - cnt[2],
         cnt[2] - cnt[3], cnt[3]]
    hw = sum(_r(h[k]) * _r(pp_ref[0, k]) for k in range(5))
    ffnn = (_r(hw + pp_ref[0, 5]) * _r(pp_ref[0, 6])) + pp_ref[0, 7]
    glog = jnp.sum(_r(q) * _r(gw_ref[...]), axis=1) + pp_ref[0, 10]
    e = jnp.exp(glog - jnp.max(glog))
    tw = e / jnp.sum(e)
    s = jnp.sum(ffnn * tw)
    out_ref[...] = jnp.reshape(
        _r(s) * _r(pp_ref[0, 8]) + pp_ref[0, 9], (1, 1, 1, 1))


def kernel(batch_queries, batch_docs, emb, gate_w, gate_b,
           ffnn_w1, ffnn_b1, ffnn_w2, ffnn_b2, out_w, out_b):
    embp = jnp.pad(emb, ((0, 0), (0, _EP - _E)))
    didx = batch_docs.reshape(-1).astype(jnp.int32)
    qpad = jnp.zeros((_B, _QPW - _Q), jnp.int32)
    qidx = jnp.concatenate(
        [batch_queries.astype(jnp.int32), qpad], axis=1).reshape(-1)
    d_emb, q_emb = _sc_gather_build()(embp, didx, qidx)
    d3 = d_emb.reshape(_B * _D, _L, _EP)
    q3 = q_emb.reshape(_B, _QPW, _EP)
    gw_row = jnp.pad(gate_w.reshape(1, _E), ((0, 0), (0, _EP - _E)))
    pp = jnp.concatenate([
        ffnn_w1.reshape(-1), ffnn_b1.reshape(-1), ffnn_w2.reshape(-1),
        ffnn_b2.reshape(-1), out_w.reshape(-1), out_b.reshape(-1),
        gate_b.reshape(-1), jnp.zeros((5,), jnp.float32)]).reshape(1, 16)
    return pl.pallas_call(
        _tc_body,
        grid=(_B, _D),
        in_specs=[
            pl.BlockSpec((1, _L, _EP), lambda b, d: (b * _D + d, 0, 0)),
            pl.BlockSpec((1, _QPW, _EP), lambda b, d: (b, 0, 0)),
            pl.BlockSpec((1, _EP), lambda b, d: (0, 0)),
            pl.BlockSpec((1, 16), lambda b, d: (0, 0)),
        ],
        out_specs=pl.BlockSpec((1, 1, 1, 1), lambda b, d: (b, d, 0, 0)),
        out_shape=jax.ShapeDtypeStruct((_B, _D, 1, 1), jnp.float32),
    )(d3, q3, gw_row, pp).reshape(_B, _D)


# speedup vs baseline: 1.3308x; 1.3308x over previous
import functools

import jax
import jax.numpy as jnp
from jax import lax
from jax.experimental import pallas as pl
from jax.experimental.pallas import tpu as pltpu
from jax.experimental.pallas import tpu_sc as plsc

_B, _D, _Q, _L = 32, 8, 15, 500
_V, _E, _NB = 100000, 300, 5
_EP = 304
_NW = 32
_DPW = (_B * _D * _L) // _NW
_CH = 80
_NCH = _DPW // _CH
_QPW = 16


@functools.cache
def _sc_gather_build():
    mesh = plsc.VectorSubcoreMesh(
        core_axis_name="c", subcore_axis_name="s", num_cores=2)

    @functools.partial(
        pl.kernel,
        mesh=mesh,
        out_type=(
            jax.ShapeDtypeStruct((_B * _D * _L, _EP), jnp.float32),
            jax.ShapeDtypeStruct((_B * _QPW, _EP), jnp.float32),
        ),
        scratch_types=[
            pltpu.VMEM((_DPW,), jnp.int32),
            pltpu.VMEM((_QPW,), jnp.int32),
            pltpu.VMEM((_CH, _EP), jnp.float32),
            pltpu.SemaphoreType.DMA,
        ],
        compiler_params=pltpu.CompilerParams(use_tc_tiling_on_sc=False),
    )
    def sc_gather(emb_hbm, didx_hbm, qidx_hbm, dout_hbm, qout_hbm,
                  didx_v, qidx_v, buf, sem):
        wid = lax.axis_index("s") * 2 + lax.axis_index("c")
        dbase = wid * _DPW
        qbase = wid * _QPW
        pltpu.sync_copy(didx_hbm.at[pl.ds(dbase, _DPW)], didx_v)
        pltpu.sync_copy(qidx_hbm.at[pl.ds(qbase, _QPW)], qidx_v)
        pltpu.async_copy(emb_hbm.at[qidx_v], buf.at[pl.ds(0, _QPW)], sem).wait()
        pltpu.sync_copy(buf.at[pl.ds(0, _QPW)], qout_hbm.at[pl.ds(qbase, _QPW)])

        def chunk(k, carry):
            off = k * _CH
            pltpu.async_copy(
                emb_hbm.at[didx_v.at[pl.ds(off, _CH)]], buf, sem).wait()
            pltpu.sync_copy(buf, dout_hbm.at[pl.ds(dbase + off, _CH)])
            return carry

        lax.fori_loop(0, _NCH, chunk, 0)

    return sc_gather


_PADBLK = 2000


def _pad_body(x_ref, o_ref):
    o_ref[:, 0:_E] = x_ref[...]
    o_ref[:, _E:_EP] = jnp.zeros((_PADBLK, _EP - _E), jnp.float32)


def _pad_table(emb):
    return pl.pallas_call(
        _pad_body,
        grid=(_V // _PADBLK,),
        in_specs=[pl.BlockSpec((_PADBLK, _E), lambda i: (i, 0))],
        out_specs=pl.BlockSpec((_PADBLK, _EP), lambda i: (i, 0)),
        out_shape=jax.ShapeDtypeStruct((_V, _EP), jnp.float32),
    )(emb)


def _tc_body(d_ref, q_ref, gw_ref, pp_ref, out_ref):
    d = d_ref[0]
    q = q_ref[0, 0:_Q, :]
    dots = lax.dot_general(
        d, q, (((1,), (1,)), ((), ())),
        preferred_element_type=jnp.float32,
        precision=lax.Precision.DEFAULT)
    dn = jnp.sqrt(jnp.sum(d * d, axis=1, keepdims=True))
    qn = jnp.sqrt(jnp.sum(q * q, axis=1))[None, :]
    denom = jnp.maximum(dn * qn, 1e-8)
    cos = jnp.clip(dots / denom, -1.0, 1.0)
    cnt = [jnp.sum((cos >= t).astype(jnp.float32), axis=0)
           for t in (-0.5, 0.0, 0.5, 1.0)]
    def _r(x):
        return x.astype(jnp.bfloat16).astype(jnp.float32)

    h = [jnp.float32(_L) - cnt[0], cnt[0] - cnt[1], cnt[1] - cnt[2],
         cnt[2] - cnt[3], cnt[3]]
    hw = sum(_r(h[k]) * _r(pp_ref[0, k]) for k in range(5))
    ffnn = (_r(hw + pp_ref[0, 5]) * _r(pp_ref[0, 6])) + pp_ref[0, 7]
    glog = jnp.sum(_r(q) * _r(gw_ref[...]), axis=1) + pp_ref[0, 10]
    e = jnp.exp(glog - jnp.max(glog))
    tw = e / jnp.sum(e)
    s = jnp.sum(ffnn * tw)
    out_ref[...] = jnp.reshape(
        _r(s) * _r(pp_ref[0, 8]) + pp_ref[0, 9], (1, 1, 1, 1))


def kernel(batch_queries, batch_docs, emb, gate_w, gate_b,
           ffnn_w1, ffnn_b1, ffnn_w2, ffnn_b2, out_w, out_b):
    embp = _pad_table(emb)
    didx = batch_docs.reshape(-1).astype(jnp.int32)
    qpad = jnp.zeros((_B, _QPW - _Q), jnp.int32)
    qidx = jnp.concatenate(
        [batch_queries.astype(jnp.int32), qpad], axis=1).reshape(-1)
    d_emb, q_emb = _sc_gather_build()(embp, didx, qidx)
    d3 = d_emb.reshape(_B * _D, _L, _EP)
    q3 = q_emb.reshape(_B, _QPW, _EP)
    gw_row = jnp.pad(gate_w.reshape(1, _E), ((0, 0), (0, _EP - _E)))
    pp = jnp.concatenate([
        ffnn_w1.reshape(-1), ffnn_b1.reshape(-1), ffnn_w2.reshape(-1),
        ffnn_b2.reshape(-1), out_w.reshape(-1), out_b.reshape(-1),
        gate_b.reshape(-1), jnp.zeros((5,), jnp.float32)]).reshape(1, 16)
    return pl.pallas_call(
        _tc_body,
        grid=(_B, _D),
        in_specs=[
            pl.BlockSpec((1, _L, _EP), lambda b, d: (b * _D + d, 0, 0)),
            pl.BlockSpec((1, _QPW, _EP), lambda b, d: (b, 0, 0)),
            pl.BlockSpec((1, _EP), lambda b, d: (0, 0)),
            pl.BlockSpec((1, 16), lambda b, d: (0, 0)),
        ],
        out_specs=pl.BlockSpec((1, 1, 1, 1), lambda b, d: (b, d, 0, 0)),
        out_shape=jax.ShapeDtypeStruct((_B, _D, 1, 1), jnp.float32),
    )(d3, q3, gw_row, pp).reshape(_B, _D)


# speedup vs baseline: 1.3728x; 1.0316x over previous
import functools

import jax
import jax.numpy as jnp
from jax import lax
from jax.experimental import pallas as pl
from jax.experimental.pallas import tpu as pltpu
from jax.experimental.pallas import tpu_sc as plsc

_B, _D, _Q, _L = 32, 8, 15, 500
_V, _E, _NB = 100000, 300, 5
_EP = 304
_NW = 32
_DPW = (_B * _D * _L) // _NW
_CH = 80
_NCH = _DPW // _CH
_QPW = 16


_NBUF = 4
_LOOKAHEAD = 2


@functools.cache
def _sc_gather_build():
    mesh = plsc.VectorSubcoreMesh(
        core_axis_name="c", subcore_axis_name="s", num_cores=2)

    @functools.partial(
        pl.kernel,
        mesh=mesh,
        out_type=(
            jax.ShapeDtypeStruct((_B * _D * _L, _EP), jnp.float32),
            jax.ShapeDtypeStruct((_B * _QPW, _EP), jnp.float32),
        ),
        scratch_types=[
            pltpu.VMEM((_DPW,), jnp.int32),
            pltpu.VMEM((_QPW,), jnp.int32),
        ] + [pltpu.VMEM((_CH, _EP), jnp.float32) for _ in range(_NBUF)]
          + [pltpu.SemaphoreType.DMA for _ in range(2 * _NBUF)],
        compiler_params=pltpu.CompilerParams(use_tc_tiling_on_sc=False),
    )
    def sc_gather(emb_hbm, didx_hbm, qidx_hbm, dout_hbm, qout_hbm,
                  didx_v, qidx_v, *bufs_and_sems):
        bufs = bufs_and_sems[:_NBUF]
        gsem = bufs_and_sems[_NBUF:2 * _NBUF]
        wsem = bufs_and_sems[2 * _NBUF:]
        wid = lax.axis_index("s") * 2 + lax.axis_index("c")
        dbase = wid * _DPW
        qbase = wid * _QPW
        pltpu.sync_copy(didx_hbm.at[pl.ds(dbase, _DPW)], didx_v)
        pltpu.sync_copy(qidx_hbm.at[pl.ds(qbase, _QPW)], qidx_v)
        pltpu.async_copy(
            emb_hbm.at[qidx_v], bufs[0].at[pl.ds(0, _QPW)], gsem[0]).wait()
        pltpu.sync_copy(
            bufs[0].at[pl.ds(0, _QPW)], qout_hbm.at[pl.ds(qbase, _QPW)])

        def g_start(k):
            b = k % _NBUF
            return pltpu.async_copy(
                emb_hbm.at[didx_v.at[pl.ds(k * _CH, _CH)]], bufs[b], gsem[b])

        gh = [None] * _NCH
        wh = [None] * _NCH
        for k in range(_LOOKAHEAD):
            gh[k] = g_start(k)
        for k in range(_NCH):
            ka = k + _LOOKAHEAD
            if ka < _NCH:
                kw = ka - _NBUF
                if kw >= 0:
                    wh[kw].wait()
                gh[ka] = g_start(ka)
            gh[k].wait()
            b = k % _NBUF
            wh[k] = pltpu.async_copy(
                bufs[b], dout_hbm.at[pl.ds(dbase + k * _CH, _CH)], wsem[b])
        for k in range(max(0, _NCH - _NBUF), _NCH):
            wh[k].wait()

    return sc_gather


_PADBLK = 2000


def _pad_body(x_ref, o_ref):
    o_ref[:, 0:_E] = x_ref[...]
    o_ref[:, _E:_EP] = jnp.zeros((_PADBLK, _EP - _E), jnp.float32)


def _pad_table(emb):
    return pl.pallas_call(
        _pad_body,
        grid=(_V // _PADBLK,),
        in_specs=[pl.BlockSpec((_PADBLK, _E), lambda i: (i, 0))],
        out_specs=pl.BlockSpec((_PADBLK, _EP), lambda i: (i, 0)),
        out_shape=jax.ShapeDtypeStruct((_V, _EP), jnp.float32),
    )(emb)


def _tc_body(d_ref, q_ref, gw_ref, pp_ref, out_ref):
    d = d_ref[0]
    q = q_ref[0, 0:_Q, :]
    dots = lax.dot_general(
        d, q, (((1,), (1,)), ((), ())),
        preferred_element_type=jnp.float32,
        precision=lax.Precision.DEFAULT)
    dn = jnp.sqrt(jnp.sum(d * d, axis=1, keepdims=True))
    qn = jnp.sqrt(jnp.sum(q * q, axis=1))[None, :]
    denom = jnp.maximum(dn * qn, 1e-8)
    cos = jnp.clip(dots / denom, -1.0, 1.0)
    cnt = [jnp.sum((cos >= t).astype(jnp.float32), axis=0)
           for t in (-0.5, 0.0, 0.5, 1.0)]
    def _r(x):
        return x.astype(jnp.bfloat16).astype(jnp.float32)

    h = [jnp.float32(_L) - cnt[0], cnt[0] - cnt[1], cnt[1] - cnt[2],
         cnt[2] - cnt[3], cnt[3]]
    hw = sum(_r(h[k]) * _r(pp_ref[0, k]) for k in range(5))
    ffnn = (_r(hw + pp_ref[0, 5]) * _r(pp_ref[0, 6])) + pp_ref[0, 7]
    glog = jnp.sum(_r(q) * _r(gw_ref[...]), axis=1) + pp_ref[0, 10]
    e = jnp.exp(glog - jnp.max(glog))
    tw = e / jnp.sum(e)
    s = jnp.sum(ffnn * tw)
    out_ref[...] = jnp.reshape(
        _r(s) * _r(pp_ref[0, 8]) + pp_ref[0, 9], (1, 1, 1, 1))


def kernel(batch_queries, batch_docs, emb, gate_w, gate_b,
           ffnn_w1, ffnn_b1, ffnn_w2, ffnn_b2, out_w, out_b):
    embp = _pad_table(emb)
    didx = batch_docs.reshape(-1).astype(jnp.int32)
    qpad = jnp.zeros((_B, _QPW - _Q), jnp.int32)
    qidx = jnp.concatenate(
        [batch_queries.astype(jnp.int32), qpad], axis=1).reshape(-1)
    d_emb, q_emb = _sc_gather_build()(embp, didx, qidx)
    d3 = d_emb.reshape(_B * _D, _L, _EP)
    q3 = q_emb.reshape(_B, _QPW, _EP)
    gw_row = jnp.pad(gate_w.reshape(1, _E), ((0, 0), (0, _EP - _E)))
    pp = jnp.concatenate([
        ffnn_w1.reshape(-1), ffnn_b1.reshape(-1), ffnn_w2.reshape(-1),
        ffnn_b2.reshape(-1), out_w.reshape(-1), out_b.reshape(-1),
        gate_b.reshape(-1), jnp.zeros((5,), jnp.float32)]).reshape(1, 16)
    return pl.pallas_call(
        _tc_body,
        grid=(_B, _D),
        in_specs=[
            pl.BlockSpec((1, _L, _EP), lambda b, d: (b * _D + d, 0, 0)),
            pl.BlockSpec((1, _QPW, _EP), lambda b, d: (b, 0, 0)),
            pl.BlockSpec((1, _EP), lambda b, d: (0, 0)),
            pl.BlockSpec((1, 16), lambda b, d: (0, 0)),
        ],
        out_specs=pl.BlockSpec((1, 1, 1, 1), lambda b, d: (b, d, 0, 0)),
        out_shape=jax.ShapeDtypeStruct((_B, _D, 1, 1), jnp.float32),
    )(d3, q3, gw_row, pp).reshape(_B, _D)


# speedup vs baseline: 1.7099x; 1.2455x over previous
import functools

import jax
import jax.numpy as jnp
from jax import lax
from jax.experimental import pallas as pl
from jax.experimental.pallas import tpu as pltpu
from jax.experimental.pallas import tpu_sc as plsc

_B, _D, _Q, _L = 32, 8, 15, 500
_V, _E, _NB = 100000, 300, 5
_EP = 304
_NW = 32
_DPW = (_B * _D * _L) // _NW
_CH = 80
_NCH = _DPW // _CH
_QPW = 16


_NBUF = 4
_LOOKAHEAD = 2


@functools.cache
def _sc_gather_build():
    mesh = plsc.VectorSubcoreMesh(
        core_axis_name="c", subcore_axis_name="s", num_cores=2)

    @functools.partial(
        pl.kernel,
        mesh=mesh,
        out_type=(
            jax.ShapeDtypeStruct((_B * _D * _L, _EP), jnp.float32),
            jax.ShapeDtypeStruct((_B * _QPW, _EP), jnp.float32),
        ),
        scratch_types=[
            pltpu.VMEM((_DPW,), jnp.int32),
            pltpu.VMEM((_QPW,), jnp.int32),
        ] + [pltpu.VMEM((_CH, _EP), jnp.float32) for _ in range(_NBUF)]
          + [pltpu.SemaphoreType.DMA for _ in range(2 * _NBUF)],
        compiler_params=pltpu.CompilerParams(use_tc_tiling_on_sc=False),
    )
    def sc_gather(emb_hbm, didx_hbm, qidx_hbm, dout_hbm, qout_hbm,
                  didx_v, qidx_v, *bufs_and_sems):
        bufs = bufs_and_sems[:_NBUF]
        gsem = bufs_and_sems[_NBUF:2 * _NBUF]
        wsem = bufs_and_sems[2 * _NBUF:]
        wid = lax.axis_index("s") * 2 + lax.axis_index("c")
        dbase = wid * _DPW
        qbase = wid * _QPW
        pltpu.sync_copy(didx_hbm.at[pl.ds(dbase, _DPW)], didx_v)
        pltpu.sync_copy(qidx_hbm.at[pl.ds(qbase, _QPW)], qidx_v)
        pltpu.async_copy(
            emb_hbm.at[qidx_v], bufs[0].at[pl.ds(0, _QPW)], gsem[0]).wait()
        pltpu.sync_copy(
            bufs[0].at[pl.ds(0, _QPW)], qout_hbm.at[pl.ds(qbase, _QPW)])

        def g_start(k):
            b = k % _NBUF
            return pltpu.async_copy(
                emb_hbm.at[didx_v.at[pl.ds(k * _CH, _CH)]], bufs[b], gsem[b])

        gh = [None] * _NCH
        wh = [None] * _NCH
        for k in range(_LOOKAHEAD):
            gh[k] = g_start(k)
        for k in range(_NCH):
            ka = k + _LOOKAHEAD
            if ka < _NCH:
                kw = ka - _NBUF
                if kw >= 0:
                    wh[kw].wait()
                gh[ka] = g_start(ka)
            gh[k].wait()
            b = k % _NBUF
            wh[k] = pltpu.async_copy(
                bufs[b], dout_hbm.at[pl.ds(dbase + k * _CH, _CH)], wsem[b])
        for k in range(max(0, _NCH - _NBUF), _NCH):
            wh[k].wait()

    return sc_gather


_PADBLK = 2000


def _pad_body(x_ref, o_ref):
    o_ref[:, 0:_E] = x_ref[...]
    o_ref[:, _E:_EP] = jnp.zeros((_PADBLK, _EP - _E), jnp.float32)


def _pad_table(emb):
    return pl.pallas_call(
        _pad_body,
        grid=(_V // _PADBLK,),
        in_specs=[pl.BlockSpec((_PADBLK, _E), lambda i: (i, 0))],
        out_specs=pl.BlockSpec((_PADBLK, _EP), lambda i: (i, 0)),
        out_shape=jax.ShapeDtypeStruct((_V, _EP), jnp.float32),
    )(emb)


def _tc_body(d_ref, q_ref, gw_ref, pp_ref, out_ref):
    d = d_ref[...]
    q = q_ref[0:_Q, :]
    dots = lax.dot_general(
        d, q, (((1,), (1,)), ((), ())),
        preferred_element_type=jnp.float32,
        precision=lax.Precision.DEFAULT)
    dn = jnp.sqrt(jnp.sum(d * d, axis=1, keepdims=True))
    qn = jnp.sqrt(jnp.sum(q * q, axis=1))[None, :]
    denom = jnp.maximum(dn * qn, 1e-8)
    cos = jnp.clip(dots / denom, -1.0, 1.0)
    def _r(x):
        return x.astype(jnp.bfloat16).astype(jnp.float32)

    glog = jnp.sum(_r(q) * _r(gw_ref[...]), axis=1) + pp_ref[0, 10]
    e = jnp.exp(glog - jnp.max(glog))
    tw = e / jnp.sum(e)
    scores = []
    for dd in range(_D):
        cs = cos[dd * _L:(dd + 1) * _L]
        cnt = [jnp.sum((cs >= t).astype(jnp.float32), axis=0)
               for t in (-0.5, 0.0, 0.5, 1.0)]
        h = [jnp.float32(_L) - cnt[0], cnt[0] - cnt[1], cnt[1] - cnt[2],
             cnt[2] - cnt[3], cnt[3]]
        hw = sum(_r(h[k]) * _r(pp_ref[0, k]) for k in range(5))
        ffnn = (_r(hw + pp_ref[0, 5]) * _r(pp_ref[0, 6])) + pp_ref[0, 7]
        s = jnp.sum(ffnn * tw)
        scores.append(_r(s) * _r(pp_ref[0, 8]) + pp_ref[0, 9])
    out_ref[...] = jnp.stack(scores).reshape(1, 1, _D)


def kernel(batch_queries, batch_docs, emb, gate_w, gate_b,
           ffnn_w1, ffnn_b1, ffnn_w2, ffnn_b2, out_w, out_b):
    embp = _pad_table(emb)
    didx = batch_docs.reshape(-1).astype(jnp.int32)
    qpad = jnp.zeros((_B, _QPW - _Q), jnp.int32)
    qidx = jnp.concatenate(
        [batch_queries.astype(jnp.int32), qpad], axis=1).reshape(-1)
    d_emb, q_emb = _sc_gather_build()(embp, didx, qidx)
    gw_row = jnp.pad(gate_w.reshape(1, _E), ((0, 0), (0, _EP - _E)))
    pp = jnp.concatenate([
        ffnn_w1.reshape(-1), ffnn_b1.reshape(-1), ffnn_w2.reshape(-1),
        ffnn_b2.reshape(-1), out_w.reshape(-1), out_b.reshape(-1),
        gate_b.reshape(-1), jnp.zeros((5,), jnp.float32)]).reshape(1, 16)
    return pl.pallas_call(
        _tc_body,
        grid=(_B,),
        in_specs=[
            pl.BlockSpec((_D * _L, _EP), lambda b: (b, 0)),
            pl.BlockSpec((_QPW, _EP), lambda b: (b, 0)),
            pl.BlockSpec((1, _EP), lambda b: (0, 0)),
            pl.BlockSpec((1, 16), lambda b: (0, 0)),
        ],
        out_specs=pl.BlockSpec((1, 1, _D), lambda b: (b, 0, 0)),
        out_shape=jax.ShapeDtypeStruct((_B, 1, _D), jnp.float32),
    )(d_emb, q_emb, gw_row, pp).reshape(_B, _D)
